# Initial kernel scaffold; baseline (speedup 1.0000x reference)
import jax
import jax.numpy as jnp
from jax.experimental import pallas as pl


def kernel(x, batch):
    raise NotImplementedError("write your pallas kernel here")


# speedup vs baseline: 4.0298x; 4.0298x over previous
import functools

import jax
import jax.numpy as jnp
from jax import lax
from jax.experimental import pallas as pl
from jax.experimental.pallas import tpu as pltpu
from jax.experimental.pallas import tpu_sc as plsc

NUM_SEGMENTS = 128
LANES = 16


def _sc_partials(n, d, num_workers, chunk, blk):
    groups_per_blk = blk // LANES
    num_blks = chunk // blk
    acc_words = NUM_SEGMENTS * d
    d_vecs = d // LANES

    mesh = plsc.VectorSubcoreMesh(core_axis_name="c", subcore_axis_name="s")

    @functools.partial(
        pl.kernel,
        mesh=mesh,
        out_type=jax.ShapeDtypeStruct((num_workers * acc_words,), jnp.float32),
        scratch_types=[
            pltpu.VMEM((chunk,), jnp.int32),
            pltpu.VMEM((blk * d,), jnp.float32),
            pltpu.VMEM((acc_words,), jnp.float32),
        ],
    )
    def sc_kernel(x_hbm, ids_hbm, out_hbm, ids_v, xbuf, acc):
        wid = lax.axis_index("c") * 16 + lax.axis_index("s")
        row0 = wid * chunk

        neg_inf = jnp.full((LANES,), -jnp.inf, dtype=jnp.float32)

        def init_body(i, carry):
            acc[pl.ds(i * LANES, LANES)] = neg_inf
            return carry

        lax.fori_loop(0, acc_words // LANES, init_body, 0)

        pltpu.sync_copy(ids_hbm.at[pl.ds(row0, chunk)], ids_v)

        def block_body(b, carry):
            pltpu.sync_copy(
                x_hbm.at[pl.ds((row0 + b * blk) * d, blk * d)], xbuf)

            def group_body(g, carry2):
                base = b * blk + g * LANES
                v = ids_v[pl.ds(base, LANES)]
                lo = v[0]
                hi = v[LANES - 1]
                gbase = g * (LANES * d)

                @pl.when(lo == hi)
                def _fast():
                    soff = lo * d
                    for j in range(d_vecs):
                        vals = [xbuf[pl.ds(gbase + r * d + j * LANES, LANES)]
                                for r in range(LANES)]
                        while len(vals) > 1:
                            vals = [jnp.maximum(vals[2 * i], vals[2 * i + 1])
                                    for i in range(len(vals) // 2)]
                        a = acc[pl.ds(soff + j * LANES, LANES)]
                        acc[pl.ds(soff + j * LANES, LANES)] = jnp.maximum(a, vals[0])

                @pl.when(lo != hi)
                def _slow():
                    for r in range(LANES):
                        s = v[r]
                        soff = s * d
                        for j in range(d_vecs):
                            a = acc[pl.ds(soff + j * LANES, LANES)]
                            xv = xbuf[pl.ds(gbase + r * d + j * LANES, LANES)]
                            acc[pl.ds(soff + j * LANES, LANES)] = jnp.maximum(a, xv)

                return carry2

            lax.fori_loop(0, groups_per_blk, group_body, carry)
            return carry

        lax.fori_loop(0, num_blks, block_body, 0)

        pltpu.sync_copy(acc, out_hbm.at[pl.ds(wid * acc_words, acc_words)])

    return sc_kernel


def _merge_body(p_ref, o_ref):
    o_ref[...] = jnp.max(p_ref[...], axis=0)


def kernel(x, batch):
    n, d = x.shape
    num_workers = 32
    chunk = n // num_workers
    blk = 400
    assert chunk % blk == 0 and blk % LANES == 0

    sc = _sc_partials(n, d, num_workers, chunk, blk)
    partials = sc(x.reshape(-1), batch)
    p3 = partials.reshape(num_workers, NUM_SEGMENTS, d)

    out = pl.pallas_call(
        _merge_body,
        out_shape=jax.ShapeDtypeStruct((NUM_SEGMENTS, d), jnp.float32),
    )(p3)
    return out
